# baseline (speedup 1.0000x reference)
import numpy as np
import jax
import jax.numpy as jnp
from jax import lax
from jax.experimental import pallas as pl
from jax.experimental.pallas import tpu as pltpu
from jax.experimental.pallas import tpu_sc as plsc

_B, _C, _H, _W = 4, 96, 384, 384
_HW = _H * _W
_K1 = int(0.5 * _HW) + 1
_N = _B * _C * _H * _W
_BH = 32
_NH = _H // _BH

_NCORE = 2
_NSUB = 16
_SLICES = 8
_CROWS = _H // _SLICES
_CVECS = _W // 16


def _make_random_mask() -> np.ndarray:
    rti = int(0.1 * _HW)
    base = jnp.concatenate([
        jnp.ones((rti,), dtype=jnp.float32),
        jnp.zeros((_HW - rti,), dtype=jnp.float32),
    ])
    keys = jax.random.split(jax.random.key(42), _B)
    rm = jax.vmap(lambda k: jax.random.permutation(k, base))(keys)
    return np.asarray(rm).reshape(_B, _H, _W)


_RMASK = _make_random_mask()


def _res_body(x_ref, y_ref, o_ref):
    o_ref[0] = jnp.sum(jnp.abs(x_ref[0] - y_ref[0]), axis=0)


def _residual_image(x, y):
    return pl.pallas_call(
        _res_body,
        grid=(_B, _NH),
        in_specs=[
            pl.BlockSpec((1, _C, _BH, _W), lambda b, h: (b, 0, h, 0)),
            pl.BlockSpec((1, _C, _BH, _W), lambda b, h: (b, 0, h, 0)),
        ],
        out_specs=pl.BlockSpec((1, _BH, _W), lambda b, h: (b, h, 0)),
        out_shape=jax.ShapeDtypeStruct((_B, _H, _W), jnp.float32),
    )(x, y)


def _sc_body(res_hbm, rm_hbm, out_hbm, vres, vrm, stage_f, smem_cnt):
    c = lax.axis_index("c")
    s = lax.axis_index("s")
    b = c * 2 + s // _SLICES
    r0 = (s % _SLICES) * _CROWS

    pltpu.sync_copy(res_hbm.at[b, pl.ds(r0, _CROWS)], vres)
    pltpu.sync_copy(rm_hbm.at[b, pl.ds(r0, _CROWS)], vrm)

    ones_f = jnp.ones((16,), jnp.float32)
    zeros_f = jnp.zeros((16,), jnp.float32)

    def count_ge(cand_f):
        cand_v = jax.lax.broadcast(cand_f, (16,))
        def row_body(r, acc):
            for cc in range(_CVECS):
                v = vres[r, pl.ds(cc * 16, 16)]
                acc = acc + jnp.where(v >= cand_v, ones_f, zeros_f)
            return acc
        acc = lax.fori_loop(0, _CROWS, row_body, zeros_f)
        tot = acc[0]
        for l in range(1, 16):
            tot = tot + acc[l]
        return tot

    leader = (s // _SLICES) * _SLICES

    def zero_body(i, _):
        smem_cnt[i] = jnp.int32(0)
        return 0
    lax.fori_loop(0, 31, zero_body, 0)
    plsc.subcore_barrier()

    def round_body(i, pfx):
        cand = pfx | (jnp.int32(1) << (jnp.int32(30) - i))
        cnt = count_ge(lax.bitcast_convert_type(cand, jnp.float32))
        plsc.fetch_and_add(smem_cnt.at[i], cnt.astype(jnp.int32),
                           subcore_id=leader)
        plsc.subcore_barrier()
        tot = plsc.fetch_and_add(smem_cnt.at[i], jnp.int32(0),
                                 subcore_id=leader)
        return jnp.where(tot >= _K1, cand, pfx)

    thre = lax.fori_loop(0, 31, round_body, jnp.int32(0))
    thre_f = lax.bitcast_convert_type(thre, jnp.float32)

    thre_v = jax.lax.broadcast(thre_f, (16,))

    def msum_row(r, acc):
        for cc in range(_CVECS):
            v = vres[r, pl.ds(cc * 16, 16)]
            m = vrm[r, pl.ds(cc * 16, 16)]
            keep = (v > thre_v) | (m > zeros_f)
            acc = acc + jnp.where(keep, v, zeros_f)
        return acc

    acc = lax.fori_loop(0, _CROWS, msum_row, zeros_f)
    stage_f[pl.ds(0, 16)] = acc
    for q in range(1, 4):
        stage_f[pl.ds(q * 16, 16)] = zeros_f
    pltpu.sync_copy(stage_f, out_hbm.at[c * _NSUB + s])


def _sc_select_and_sum(res, rmask):
    mesh = plsc.VectorSubcoreMesh(core_axis_name="c", subcore_axis_name="s")
    f = pl.kernel(
        _sc_body,
        out_type=jax.ShapeDtypeStruct((_NCORE * _NSUB, 64), jnp.float32),
        mesh=mesh,
        scratch_types=[
            pltpu.VMEM((_CROWS, _W), jnp.float32),
            pltpu.VMEM((_CROWS, _W), jnp.float32),
            pltpu.VMEM((64,), jnp.float32),
            pltpu.SMEM((32,), jnp.int32),
        ],
    )
    return f(res, rmask)


def kernel(x, y):
    res = _residual_image(x, y)
    partials = _sc_select_and_sum(res, jnp.asarray(_RMASK))
    return jnp.sum(partials) / jnp.float32(_N)
